# baseline (speedup 1.0000x reference)
import functools

import jax
import jax.numpy as jnp
from jax import lax
from jax.experimental import pallas as pl
from jax.experimental.pallas import tpu as pltpu
from jax.experimental.pallas import tpu_sc as plsc

_NC = 2
_NS = 16
_NW = _NC * _NS


def _make_gather(fields: int, batch: int, dim: int):
    mesh = plsc.VectorSubcoreMesh(core_axis_name="c", subcore_axis_name="s")
    nb = batch // _NW

    @functools.partial(
        pl.kernel,
        mesh=mesh,
        out_type=jax.ShapeDtypeStruct((fields, dim, batch), jnp.float32),
        compiler_params=pltpu.CompilerParams(use_tc_tiling_on_sc=False,
                                             needs_layout_passes=False),
        scratch_types=[
            pltpu.VMEM((fields, nb), jnp.int32),
            pltpu.VMEM((fields * nb, dim), jnp.float32),
            pltpu.VMEM((dim, nb), jnp.float32),
            pltpu.VMEM((dim, nb), jnp.float32),
            pltpu.SemaphoreType.DMA,
            pltpu.SemaphoreType.DMA,
            pltpu.SemaphoreType.DMA,
        ],
    )
    def gather_kernel(idx_hbm, table_hbm, out_hbm, idx_v, rows_v,
                      slab0, slab1, gsem, wsem0, wsem1):
        wid = lax.axis_index("s") * _NC + lax.axis_index("c")
        b0 = wid * nb
        for f in range(fields):
            pltpu.async_copy(idx_hbm.at[f * _NW + wid], idx_v.at[f], gsem)
        pltpu.make_async_copy(idx_hbm.at[pl.ds(0, fields)], idx_v,
                              gsem).wait()
        for f in range(fields):
            pltpu.async_copy(table_hbm.at[idx_v.at[f]],
                             rows_v.at[pl.ds(f * nb, nb)], gsem)
        pltpu.make_async_copy(table_hbm.at[pl.ds(0, fields * nb)], rows_v,
                              gsem).wait()
        slabs = (slab0, slab1)
        wsems = (wsem0, wsem1)
        lanes = lax.iota(jnp.int32, 16)
        rots = [jnp.bitwise_and(lanes + k, 15) for k in range(16)]

        def transpose_field(f, slab):
            def g_body(g, _):
                row_d = f * nb + g * 16 + lanes
                for h in range(dim // 16):
                    for k in range(16):
                        col = rots[k] + (h * 16)
                        vals = plsc.load_gather(rows_v, [row_d, col])
                        plsc.store_scatter(slab, [col, g * 16 + lanes], vals)
                return 0

            lax.fori_loop(0, nb // 16, g_body, 0)

        def drain_wb(slab, wsem):
            pltpu.make_async_copy(out_hbm.at[0, :, pl.ds(0, nb)], slab,
                                  wsem).wait()

        def do_field(f, p):
            transpose_field(f, slabs[p])
            pltpu.async_copy(slabs[p], out_hbm.at[f, :, pl.ds(b0, nb)],
                             wsems[p])

        do_field(0, 0)
        do_field(1, 1)

        def pair_body(t, _):
            for q in range(2):
                drain_wb(slabs[q], wsems[q])
                do_field(2 * t + q, q)
            return 0

        lax.fori_loop(1, fields // 2, pair_body, 0)
        drain_wb(slabs[0], wsems[0])
        drain_wb(slabs[1], wsems[1])

    return gather_kernel


def kernel(inputs, embedding):
    batch, fields = inputs.shape
    vocab, dim = embedding.shape
    assert batch % _NW == 0
    idx_t = inputs.T.reshape(fields * batch // 128, 128).astype(jnp.int32)
    out_t = _make_gather(fields, batch, dim)(idx_t, embedding)
    return out_t.transpose(2, 0, 1)
